# baseline (speedup 1.0000x reference)
import functools

import jax
import jax.numpy as jnp
from jax import lax
from jax.experimental import pallas as pl
from jax.experimental.pallas import tpu as pltpu
from jax.experimental.pallas import tpu_sc as plsc

_NC = 2
_NS = 16
_LANES = 16
_CHUNK = 80


def _sc_body(n_chunks, rows_per_tile, nvec,
             x_hbm, src_hbm, dst_hbm, adj_hbm, partial_hbm,
             src_v, dst_v, adj_v, bufs, acc, gsems):
    c = lax.axis_index("c")
    s = lax.axis_index("s")
    wid = c * _NS + s

    zero = jnp.zeros((_LANES,), jnp.float32)
    for r in range(8):
        for k in range(nvec):
            bufs[0][r, pl.ds(k * _LANES, _LANES)] = zero

    def zcopy(b, carry):
        pltpu.sync_copy(bufs[0].at[pl.ds(0, 8)],
                        acc.at[pl.ds(s * rows_per_tile + b * 8, 8)])
        return carry

    lax.fori_loop(0, rows_per_tile // 8, zcopy, 0)
    plsc.subcore_barrier()

    base = wid * n_chunks * _CHUNK

    def load_idx(i, ci):
        off = base + ci * _CHUNK
        pltpu.sync_copy(src_hbm.at[pl.ds(off, _CHUNK)], src_v[i])
        pltpu.sync_copy(dst_hbm.at[pl.ds(off, _CHUNK)], dst_v[i])
        pltpu.sync_copy(adj_hbm.at[pl.ds(off, _CHUNK)], adj_v[i])

    def start_gather(i):
        pltpu.async_copy(x_hbm.at[src_v[i]], bufs[i], gsems[i])

    def wait_gather(i):
        pltpu.make_async_copy(x_hbm.at[pl.ds(0, _CHUNK)], bufs[i],
                              gsems[i]).wait()

    def compute_scale(i):
        buf = bufs[i]

        def group(g, ecarry):
            a16 = adj_v[i][pl.ds(g * _LANES, _LANES)]
            for j in range(_LANES):
                av = jnp.full((_LANES,), a16[j], jnp.float32)
                row = g * _LANES + j
                for k in range(nvec):
                    sl = pl.ds(k * _LANES, _LANES)
                    buf[row, sl] = buf[row, sl] * av
            return ecarry

        lax.fori_loop(0, _CHUNK // _LANES, group, 0)

    def process(i, ci):
        wait_gather(i)
        compute_scale(i)
        pltpu.sync_copy(bufs[i], acc.at[dst_v[i]], add=True)
        nb = (i + 2) % 3

        @pl.when(ci + 2 < n_chunks)
        def _():
            load_idx(nb, ci + 2)
            start_gather(nb)

    load_idx(0, 0)
    start_gather(0)
    load_idx(1, 1)
    start_gather(1)

    def chunk_body(ci, carry):
        for i in range(3):
            @pl.when(ci % 3 == i)
            def _(i=i):
                process(i, ci)

        return carry

    lax.fori_loop(0, n_chunks, chunk_body, 0)
    plsc.subcore_barrier()

    r0 = s * rows_per_tile
    pltpu.sync_copy(acc.at[pl.ds(r0, rows_per_tile)],
                    partial_hbm.at[c, pl.ds(r0, rows_per_tile)])


@jax.jit
def _sc_spmm(x, src, dst, adj):
    n, d = x.shape
    e = adj.shape[0]
    nw = _NC * _NS
    n_chunks = e // (nw * _CHUNK)
    align = _NS * 8
    n_pad = ((n + align - 1) // align) * align
    rows_per_tile = n_pad // _NS

    mesh = plsc.VectorSubcoreMesh(core_axis_name="c", subcore_axis_name="s")
    body = functools.partial(_sc_body, n_chunks, rows_per_tile, d // _LANES)

    def wrapped(x_hbm, src_hbm, dst_hbm, adj_hbm, partial_hbm,
                s0, s1, s2, d0, d1, d2, a0, a1, a2, b0, b1, b2,
                acc, g0, g1, g2):
        body(x_hbm, src_hbm, dst_hbm, adj_hbm, partial_hbm,
             (s0, s1, s2), (d0, d1, d2), (a0, a1, a2), (b0, b1, b2),
             acc, (g0, g1, g2))

    f = pl.kernel(
        wrapped,
        out_type=jax.ShapeDtypeStruct((_NC, n_pad, d), jnp.float32),
        mesh=mesh,
        scratch_types=(
            [pltpu.VMEM((_CHUNK,), jnp.int32)] * 6
            + [pltpu.VMEM((_CHUNK,), jnp.float32)] * 3
            + [pltpu.VMEM((_CHUNK, d), jnp.float32)] * 3
            + [pltpu.VMEM_SHARED((n_pad, d), jnp.float32)]
            + [pltpu.SemaphoreType.DMA] * 3
        ),
    )
    return f(x, src, dst, adj)


def _combine_body(p_ref, w_ref, o_ref):
    o_ref[...] = (p_ref[0] + p_ref[1]) * w_ref[...]


def _combine(partial, w2d, n):
    _, n_pad, d = partial.shape
    blk = 1000 if n % 1000 == 0 else n
    grid_r = n // blk
    return pl.pallas_call(
        _combine_body,
        grid=(grid_r,),
        in_specs=[
            pl.BlockSpec((_NC, blk, d), lambda i: (0, i, 0)),
            pl.BlockSpec((1, d), lambda i: (0, 0)),
        ],
        out_specs=pl.BlockSpec((blk, d), lambda i: (i, 0)),
        out_shape=jax.ShapeDtypeStruct((n, d), jnp.float32),
    )(partial, w2d)


def kernel(x, edge_index, adj_values, W):
    n, d = x.shape
    dst = edge_index[0]
    src = edge_index[1]
    partial = _sc_spmm(x, src, dst, adj_values)
    return _combine(partial, W.reshape(1, d), n)


# speedup vs baseline: 1.0027x; 1.0027x over previous
import functools

import jax
import jax.numpy as jnp
from jax import lax
from jax.experimental import pallas as pl
from jax.experimental.pallas import tpu as pltpu
from jax.experimental.pallas import tpu_sc as plsc

_NC = 2
_NS = 16
_LANES = 16
_CHUNK = 80


def _sc_body(n_chunks, rows_per_tile, nvec,
             x_hbm, src_hbm, dst_hbm, adj_hbm, partial_hbm,
             src_v, dst_v, adj_v, bufs, acc, gsems):
    c = lax.axis_index("c")
    s = lax.axis_index("s")
    wid = c * _NS + s

    zero = jnp.zeros((_LANES,), jnp.float32)
    for r in range(8):
        for k in range(nvec):
            bufs[0][r, pl.ds(k * _LANES, _LANES)] = zero

    def zcopy(b, carry):
        pltpu.sync_copy(bufs[0].at[pl.ds(0, 8)],
                        acc.at[pl.ds(s * rows_per_tile + b * 8, 8)])
        return carry

    lax.fori_loop(0, rows_per_tile // 8, zcopy, 0)
    plsc.subcore_barrier()

    base = wid * n_chunks * _CHUNK

    def load_idx(i, ci):
        off = base + ci * _CHUNK
        pltpu.sync_copy(src_hbm.at[pl.ds(off, _CHUNK)], src_v[i])
        pltpu.sync_copy(dst_hbm.at[pl.ds(off, _CHUNK)], dst_v[i])
        pltpu.sync_copy(adj_hbm.at[pl.ds(off, _CHUNK)], adj_v[i])

    def start_gather(i):
        pltpu.async_copy(x_hbm.at[src_v[i]], bufs[i], gsems[i])

    def wait_gather(i):
        pltpu.make_async_copy(x_hbm.at[pl.ds(0, _CHUNK)], bufs[i],
                              gsems[i]).wait()

    def compute_scale(i):
        buf = bufs[i]

        def group(g, ecarry):
            a16 = adj_v[i][pl.ds(g * _LANES, _LANES)]
            for j in range(_LANES):
                av = jnp.full((_LANES,), a16[j], jnp.float32)
                row = g * _LANES + j
                for k in range(nvec):
                    sl = pl.ds(k * _LANES, _LANES)
                    buf[row, sl] = buf[row, sl] * av
            return ecarry

        lax.fori_loop(0, _CHUNK // _LANES, group, 0)

    def process(i, ci):
        wait_gather(i)
        compute_scale(i)
        pltpu.sync_copy(bufs[i], acc.at[dst_v[i]], add=True)

        @pl.when(ci + 2 < n_chunks)
        def _():
            load_idx(i, ci + 2)
            start_gather(i)

    load_idx(0, 0)
    start_gather(0)
    load_idx(1, 1)
    start_gather(1)

    def chunk_body(ci, carry):
        @pl.when(ci % 2 == 0)
        def _():
            process(0, ci)

        @pl.when(ci % 2 == 1)
        def _():
            process(1, ci)

        return carry

    lax.fori_loop(0, n_chunks, chunk_body, 0)
    plsc.subcore_barrier()

    r0 = s * rows_per_tile
    pltpu.sync_copy(acc.at[pl.ds(r0, rows_per_tile)],
                    partial_hbm.at[c, pl.ds(r0, rows_per_tile)])


@jax.jit
def _sc_spmm(x, src, dst, adj):
    n, d = x.shape
    e = adj.shape[0]
    nw = _NC * _NS
    n_chunks = e // (nw * _CHUNK)
    align = _NS * 8
    n_pad = ((n + align - 1) // align) * align
    rows_per_tile = n_pad // _NS

    mesh = plsc.VectorSubcoreMesh(core_axis_name="c", subcore_axis_name="s")
    body = functools.partial(_sc_body, n_chunks, rows_per_tile, d // _LANES)

    def wrapped(x_hbm, src_hbm, dst_hbm, adj_hbm, partial_hbm,
                s0, s1, d0, d1, a0, a1, b0, b1, acc, g0, g1):
        body(x_hbm, src_hbm, dst_hbm, adj_hbm, partial_hbm,
             (s0, s1), (d0, d1), (a0, a1), (b0, b1), acc, (g0, g1))

    f = pl.kernel(
        wrapped,
        out_type=jax.ShapeDtypeStruct((_NC, n_pad, d), jnp.float32),
        mesh=mesh,
        scratch_types=[
            pltpu.VMEM((_CHUNK,), jnp.int32),
            pltpu.VMEM((_CHUNK,), jnp.int32),
            pltpu.VMEM((_CHUNK,), jnp.int32),
            pltpu.VMEM((_CHUNK,), jnp.int32),
            pltpu.VMEM((_CHUNK,), jnp.float32),
            pltpu.VMEM((_CHUNK,), jnp.float32),
            pltpu.VMEM((_CHUNK, d), jnp.float32),
            pltpu.VMEM((_CHUNK, d), jnp.float32),
            pltpu.VMEM_SHARED((n_pad, d), jnp.float32),
            pltpu.SemaphoreType.DMA,
            pltpu.SemaphoreType.DMA,
        ],
    )
    return f(x, src, dst, adj)


def _combine_body(p_ref, w_ref, o_ref):
    o_ref[...] = (p_ref[0] + p_ref[1]) * w_ref[...]


def _combine(partial, w2d, n):
    _, n_pad, d = partial.shape
    blk = 1000 if n % 1000 == 0 else n
    grid_r = n // blk
    return pl.pallas_call(
        _combine_body,
        grid=(grid_r,),
        in_specs=[
            pl.BlockSpec((_NC, blk, d), lambda i: (0, i, 0)),
            pl.BlockSpec((1, d), lambda i: (0, 0)),
        ],
        out_specs=pl.BlockSpec((blk, d), lambda i: (i, 0)),
        out_shape=jax.ShapeDtypeStruct((n, d), jnp.float32),
    )(partial, w2d)


def kernel(x, edge_index, adj_values, W):
    n, d = x.shape
    dst = edge_index[0]
    src = edge_index[1]
    partial = _sc_spmm(x, src, dst, adj_values)
    return _combine(partial, W.reshape(1, d), n)
